# baseline (speedup 1.0000x reference)
import jax
import jax.numpy as jnp
from jax import lax
from jax.experimental import pallas as pl
from jax.experimental.pallas import tpu as pltpu
from jax.experimental.pallas import tpu_sc as plsc

_NC, _NS = 2, 16
_NW = _NC * _NS
_CH = 32 * 1024


def _sc_body(xf_hbm, pf_hbm, out_hbm, xbuf, posbuf):
    batch, flat = xf_hbm.shape
    cid = lax.axis_index("c")
    sid = lax.axis_index("s")
    wid = sid * _NC + cid
    elems_per_tile = flat // _NW
    base = wid * elems_per_tile
    for ck in range(elems_per_tile // _CH):
        off = base + ck * _CH
        pltpu.sync_copy(pf_hbm.at[pl.ds(off, _CH)], posbuf)
        for b in range(batch):
            pltpu.sync_copy(xf_hbm.at[b, pl.ds(off, _CH)], xbuf)

            @plsc.parallel_loop(0, _CH, step=16, unroll=8)
            def _(i):
                xbuf[pl.ds(i, 16)] = xbuf[pl.ds(i, 16)] + posbuf[pl.ds(i, 16)]

            pltpu.sync_copy(xbuf, out_hbm.at[b, pl.ds(off, _CH)])


def kernel(x, pos_table):
    batch, seq_len, d_model = x.shape
    xf = x.reshape(batch, seq_len * d_model)
    pf = pos_table.reshape(-1)
    k = pl.kernel(
        _sc_body,
        out_type=jax.ShapeDtypeStruct((batch, seq_len * d_model), x.dtype),
        mesh=plsc.VectorSubcoreMesh(core_axis_name="c", subcore_axis_name="s"),
        scratch_types=[
            pltpu.VMEM((_CH,), jnp.float32),
            pltpu.VMEM((_CH,), jnp.float32),
        ],
    )
    return k(xf, pf).reshape(x.shape)


# speedup vs baseline: 5.3867x; 5.3867x over previous
import jax
import jax.numpy as jnp
from jax.experimental import pallas as pl


def _add_kernel(x_ref, pos_ref, out_ref):
    out_ref[...] = x_ref[...] + pos_ref[...]


def kernel(x, pos_table):
    batch, seq_len, d_model = x.shape
    blk_s = 2048
    grid = (seq_len // blk_s, batch)
    return pl.pallas_call(
        _add_kernel,
        grid=grid,
        in_specs=[
            pl.BlockSpec((1, blk_s, d_model), lambda s, b: (b, s, 0)),
            pl.BlockSpec((blk_s, d_model), lambda s, b: (s, 0)),
        ],
        out_specs=pl.BlockSpec((1, blk_s, d_model), lambda s, b: (b, s, 0)),
        out_shape=jax.ShapeDtypeStruct(x.shape, x.dtype),
    )(x, pos_table)
